# baseline (speedup 1.0000x reference)
import jax
import jax.numpy as jnp
from jax import lax
from jax.experimental import pallas as pl
from jax.experimental.pallas import tpu as pltpu
from jax.experimental.pallas import tpu_sc as plsc

_DIM = 64
_ROW_BLOCK = 512
_ETP = 2048
_RTP = 1024


def _log_sigmoid(x):
    return jnp.minimum(x, 0.0) - jnp.log(1.0 + jnp.exp(-jnp.abs(x)))


def _make_transpose_pack(n_rows, c):
    grid = (n_rows + c - 1) // c
    half = c // 2

    def body(in_ref, out_ref):
        x = in_ref[...].reshape(_DIM, c)
        ident = jnp.eye(_DIM, dtype=jnp.float32)
        xt = lax.dot_general(x, ident, (((0,), (0,)), ((), ())),
                             preferred_element_type=jnp.float32,
                             precision=lax.Precision.HIGHEST)
        out_ref[...] = jnp.concatenate([xt[:half], xt[half:]], axis=1)

    return pl.pallas_call(
        body,
        grid=(grid,),
        in_specs=[pl.BlockSpec((8, 8, c), lambda i: (0, 0, i))],
        out_specs=pl.BlockSpec((half, 2 * _DIM), lambda i: (i, 0)),
        out_shape=jax.ShapeDtypeStruct((grid * half, 2 * _DIM), jnp.float32),
    )


def _make_sc_gather(n_ent_rows, n_rel_rows, e_shift, r_shift):
    info = plsc.get_sparse_core_info()
    nc, ns = info.num_cores, info.num_subcores
    nw = nc * ns
    ec = n_ent_rows // 128
    rc = n_rel_rows // 128
    assert ec * 128 == n_ent_rows and rc * 128 == n_rel_rows
    mesh = plsc.VectorSubcoreMesh(core_axis_name="c", subcore_axis_name="s")

    def gather_chunk(idx_hbm, src_hbm, out_hbm, idx_v, qidx_v, wide_v,
                     sem, c0, shift):
        c0 = pl.multiple_of(c0, 128)
        hmask = (1 << (shift - 1)) - 1
        pltpu.sync_copy(idx_hbm.at[pl.ds(c0, 128)], idx_v)
        for g in range(8):
            r = idx_v[pl.ds(g * 16, 16)]
            qidx_v[pl.ds(g * 16, 16)] = ((r >> shift) << (shift - 1)) | (r & hmask)
        pltpu.async_copy(src_hbm.at[qidx_v], wide_v, sem).wait()
        pltpu.sync_copy(wide_v, out_hbm.at[pl.ds(c0, 128)])

    def body(ent_hbm, rel_hbm, eidx_hbm, ridx_hbm, ent_out, rel_out,
             idx_v, qidx_v, wide_v, sem):
        wid = lax.axis_index("s") * nc + lax.axis_index("c")
        for t in range((ec + nw - 1) // nw):
            cid = lax.rem(t * nw + wid, ec)
            gather_chunk(eidx_hbm, ent_hbm, ent_out, idx_v, qidx_v, wide_v,
                         sem, cid * 128, e_shift)
        for t in range((rc + nw - 1) // nw):
            cid = lax.rem(t * nw + wid, rc)
            gather_chunk(ridx_hbm, rel_hbm, rel_out, idx_v, qidx_v, wide_v,
                         sem, cid * 128, r_shift)

    return pl.kernel(
        body,
        out_type=(jax.ShapeDtypeStruct((n_ent_rows, 2 * _DIM), jnp.float32),
                  jax.ShapeDtypeStruct((n_rel_rows, 2 * _DIM), jnp.float32)),
        mesh=mesh,
        scratch_types=[
            pltpu.VMEM((128,), jnp.int32),
            pltpu.VMEM((128,), jnp.int32),
            pltpu.VMEM((128, 2 * _DIM), jnp.float32),
            pltpu.SemaphoreType.DMA,
        ],
        compiler_params=pltpu.CompilerParams(use_tc_tiling_on_sc=True,
                                             needs_layout_passes=False),
    )


def _make_tc_score(n_pos, n_neg, interpret=False):
    b = _ROW_BLOCK
    grid = n_pos // b
    t_off = n_pos // b
    neg_blk_idx = (2 * n_pos) // n_neg

    def sel(wide, p):
        return wide[:, :_DIM] + (wide[:, _DIM:] - wide[:, :_DIM]) * p

    def body(h_ref, t_ref, neg_ref, r_ref, ph_ref, pt_ref, pn_ref, pr_ref,
             w_ref, ns_ref, pos_ref, loss_ref):
        i = pl.program_id(0)
        h = sel(h_ref[...], ph_ref[...])
        t = sel(t_ref[...], pt_ref[...])
        neg = sel(neg_ref[...], pn_ref[...])
        r = sel(r_ref[...], pr_ref[...])
        hr = h * r
        pos = jnp.sum(hr * t, axis=1)
        pos_ref[...] = pos
        s = lax.dot_general(hr, neg, (((1,), (1,)), ((), ())),
                            preferred_element_type=jnp.float32,
                            precision=lax.Precision.HIGHEST)
        ns_ref[...] = s
        pos_l = _log_sigmoid(pos)
        neg_l = jnp.mean(_log_sigmoid(-s), axis=1)
        part = jnp.sum(w_ref[...] * (pos_l + neg_l))

        @pl.when(i == 0)
        def _init():
            loss_ref[0, 0] = 0.0

        loss_ref[0, 0] -= part

    wd = 2 * _DIM
    return pl.pallas_call(
        body,
        grid=(grid,),
        in_specs=[
            pl.BlockSpec((b, wd), lambda i: (i, 0)),
            pl.BlockSpec((b, wd), lambda i: (i + t_off, 0)),
            pl.BlockSpec((n_neg, wd), lambda i: (neg_blk_idx, 0)),
            pl.BlockSpec((b, wd), lambda i: (i, 0)),
            pl.BlockSpec((b, 1), lambda i: (i, 0)),
            pl.BlockSpec((b, 1), lambda i: (i + t_off, 0)),
            pl.BlockSpec((n_neg, 1), lambda i: (neg_blk_idx, 0)),
            pl.BlockSpec((b, 1), lambda i: (i, 0)),
            pl.BlockSpec((b,), lambda i: (i,)),
        ],
        out_specs=[
            pl.BlockSpec((b, n_neg), lambda i: (i, 0)),
            pl.BlockSpec((b,), lambda i: (i,)),
            pl.BlockSpec(memory_space=pltpu.SMEM),
        ],
        out_shape=[
            jax.ShapeDtypeStruct((n_pos, n_neg), jnp.float32),
            jax.ShapeDtypeStruct((n_pos,), jnp.float32),
            jax.ShapeDtypeStruct((1, 1), jnp.float32),
        ],
        interpret=interpret,
    )


def kernel(head, relation, tail, negative, triple_weight,
           entity_embedding, relation_embedding):
    n_pos = head.size
    n_neg = negative.size
    n_ent = entity_embedding.shape[0]
    n_rel = relation_embedding.shape[0]
    h_idx = head.reshape(-1)
    r_idx = relation.reshape(-1)
    t_idx = tail.reshape(-1)
    n_idx = negative.reshape(-1)
    ent_idx = jnp.concatenate([h_idx, t_idx, n_idx])

    e3 = entity_embedding.T.reshape(8, 8, n_ent)
    r3 = relation_embedding.T.reshape(8, 8, n_rel)
    epacked = _make_transpose_pack(n_ent, _ETP)(e3)
    rpacked = _make_transpose_pack(n_rel, _RTP)(
        jnp.pad(r3, ((0, 0), (0, 0), (0, _RTP - n_rel))))

    ent_wide, rel_wide = _make_sc_gather(
        ent_idx.shape[0], n_pos, _ETP.bit_length() - 1, _RTP.bit_length() - 1)(
        epacked, rpacked, ent_idx, r_idx)

    ep = ((ent_idx >> (_ETP.bit_length() - 2)) & 1).astype(jnp.float32)
    rp = ((r_idx >> (_RTP.bit_length() - 2)) & 1).astype(jnp.float32)
    ep = ep.reshape(-1, 1)
    rp = rp.reshape(-1, 1)

    ns, pos, loss = _make_tc_score(n_pos, n_neg)(
        ent_wide, ent_wide, ent_wide, rel_wide, ep, ep, ep, rp, triple_weight)
    return loss[0, 0], pos, ns


# speedup vs baseline: 1.6909x; 1.6909x over previous
import jax
import jax.numpy as jnp
from jax import lax
from jax.experimental import pallas as pl
from jax.experimental.pallas import tpu as pltpu
from jax.experimental.pallas import tpu_sc as plsc

_DIM = 64
_ROW_BLOCK = 512
_ETP = 4096
_RTP = 1024


def _log_sigmoid(x):
    return jnp.minimum(x, 0.0) - jnp.log(1.0 + jnp.exp(-jnp.abs(x)))


def _make_transpose_pack(n_rows, c):
    grid = (n_rows + c - 1) // c
    half = c // 2

    def body(in_ref, out_ref):
        x = in_ref[...].reshape(_DIM, c)
        out_ref[:, :_DIM] = x[:, :half].T
        out_ref[:, _DIM:] = x[:, half:].T

    return pl.pallas_call(
        body,
        grid=(grid,),
        in_specs=[pl.BlockSpec((8, 8, c), lambda i: (0, 0, i))],
        out_specs=pl.BlockSpec((half, 2 * _DIM), lambda i: (i, 0)),
        out_shape=jax.ShapeDtypeStruct((grid * half, 2 * _DIM), jnp.float32),
    )


def _make_sc_gather(n_ent_rows, n_rel_rows, e_shift, r_shift):
    info = plsc.get_sparse_core_info()
    nc, ns = info.num_cores, info.num_subcores
    nw = nc * ns
    ec = n_ent_rows // 128
    rc = n_rel_rows // 128
    assert ec * 128 == n_ent_rows and rc * 128 == n_rel_rows
    mesh = plsc.VectorSubcoreMesh(core_axis_name="c", subcore_axis_name="s")

    def gather_chunk(idx_hbm, src_hbm, out_hbm, idx_v, qidx_v, wide_v,
                     sem, c0, shift):
        c0 = pl.multiple_of(c0, 128)
        hmask = (1 << (shift - 1)) - 1
        pltpu.sync_copy(idx_hbm.at[pl.ds(c0, 128)], idx_v)
        for g in range(8):
            r = idx_v[pl.ds(g * 16, 16)]
            qidx_v[pl.ds(g * 16, 16)] = ((r >> shift) << (shift - 1)) | (r & hmask)
        pltpu.async_copy(src_hbm.at[qidx_v], wide_v, sem).wait()
        pltpu.sync_copy(wide_v, out_hbm.at[pl.ds(c0, 128)])

    def body(ent_hbm, rel_hbm, eidx_hbm, ridx_hbm, ent_out, rel_out,
             idx_v, qidx_v, wide_v, sem):
        wid = lax.axis_index("s") * nc + lax.axis_index("c")
        for t in range((ec + nw - 1) // nw):
            cid = lax.rem(t * nw + wid, ec)
            gather_chunk(eidx_hbm, ent_hbm, ent_out, idx_v, qidx_v, wide_v,
                         sem, cid * 128, e_shift)
        for t in range((rc + nw - 1) // nw):
            cid = lax.rem(t * nw + wid, rc)
            gather_chunk(ridx_hbm, rel_hbm, rel_out, idx_v, qidx_v, wide_v,
                         sem, cid * 128, r_shift)

    return pl.kernel(
        body,
        out_type=(jax.ShapeDtypeStruct((n_ent_rows, 2 * _DIM), jnp.float32),
                  jax.ShapeDtypeStruct((n_rel_rows, 2 * _DIM), jnp.float32)),
        mesh=mesh,
        scratch_types=[
            pltpu.VMEM((128,), jnp.int32),
            pltpu.VMEM((128,), jnp.int32),
            pltpu.VMEM((128, 2 * _DIM), jnp.float32),
            pltpu.SemaphoreType.DMA,
        ],
        compiler_params=pltpu.CompilerParams(use_tc_tiling_on_sc=True,
                                             needs_layout_passes=False),
    )


def _make_tc_score(n_pos, n_neg, interpret=False):
    b = _ROW_BLOCK
    grid = n_pos // b
    t_off = n_pos // b
    neg_blk_idx = (2 * n_pos) // n_neg

    def sel(wide, p):
        return wide[:, :_DIM] + (wide[:, _DIM:] - wide[:, :_DIM]) * p

    def body(h_ref, t_ref, neg_ref, r_ref, ph_ref, pt_ref, pn_ref, pr_ref,
             w_ref, ns_ref, pos_ref, loss_ref):
        i = pl.program_id(0)
        h = sel(h_ref[...], ph_ref[...])
        t = sel(t_ref[...], pt_ref[...])
        neg = sel(neg_ref[...], pn_ref[...])
        r = sel(r_ref[...], pr_ref[...])
        hr = h * r
        pos = jnp.sum(hr * t, axis=1)
        pos_ref[...] = pos
        s = lax.dot_general(hr, neg, (((1,), (1,)), ((), ())),
                            preferred_element_type=jnp.float32,
                            precision=lax.Precision.HIGHEST)
        ns_ref[...] = s
        pos_l = _log_sigmoid(pos)
        neg_l = jnp.mean(_log_sigmoid(-s), axis=1)
        part = jnp.sum(w_ref[...] * (pos_l + neg_l))

        @pl.when(i == 0)
        def _init():
            loss_ref[0, 0] = 0.0

        loss_ref[0, 0] -= part

    wd = 2 * _DIM
    return pl.pallas_call(
        body,
        grid=(grid,),
        in_specs=[
            pl.BlockSpec((b, wd), lambda i: (i, 0)),
            pl.BlockSpec((b, wd), lambda i: (i + t_off, 0)),
            pl.BlockSpec((n_neg, wd), lambda i: (neg_blk_idx, 0)),
            pl.BlockSpec((b, wd), lambda i: (i, 0)),
            pl.BlockSpec((b, 1), lambda i: (i, 0)),
            pl.BlockSpec((b, 1), lambda i: (i + t_off, 0)),
            pl.BlockSpec((n_neg, 1), lambda i: (neg_blk_idx, 0)),
            pl.BlockSpec((b, 1), lambda i: (i, 0)),
            pl.BlockSpec((b,), lambda i: (i,)),
        ],
        out_specs=[
            pl.BlockSpec((b, n_neg), lambda i: (i, 0)),
            pl.BlockSpec((b,), lambda i: (i,)),
            pl.BlockSpec(memory_space=pltpu.SMEM),
        ],
        out_shape=[
            jax.ShapeDtypeStruct((n_pos, n_neg), jnp.float32),
            jax.ShapeDtypeStruct((n_pos,), jnp.float32),
            jax.ShapeDtypeStruct((1, 1), jnp.float32),
        ],
        interpret=interpret,
    )


def kernel(head, relation, tail, negative, triple_weight,
           entity_embedding, relation_embedding):
    n_pos = head.size
    n_neg = negative.size
    n_ent = entity_embedding.shape[0]
    n_rel = relation_embedding.shape[0]
    h_idx = head.reshape(-1)
    r_idx = relation.reshape(-1)
    t_idx = tail.reshape(-1)
    n_idx = negative.reshape(-1)
    ent_idx = jnp.concatenate([h_idx, t_idx, n_idx])

    e3 = entity_embedding.T.reshape(8, 8, n_ent)
    r3 = relation_embedding.T.reshape(8, 8, n_rel)
    epacked = _make_transpose_pack(n_ent, _ETP)(e3)
    rpacked = _make_transpose_pack(n_rel, _RTP)(
        jnp.pad(r3, ((0, 0), (0, 0), (0, _RTP - n_rel))))

    ent_wide, rel_wide = _make_sc_gather(
        ent_idx.shape[0], n_pos, _ETP.bit_length() - 1, _RTP.bit_length() - 1)(
        epacked, rpacked, ent_idx, r_idx)

    ep = ((ent_idx >> (_ETP.bit_length() - 2)) & 1).astype(jnp.float32)
    rp = ((r_idx >> (_RTP.bit_length() - 2)) & 1).astype(jnp.float32)
    ep = ep.reshape(-1, 1)
    rp = rp.reshape(-1, 1)

    ns, pos, loss = _make_tc_score(n_pos, n_neg)(
        ent_wide, ent_wide, ent_wide, rel_wide, ep, ep, ep, rp, triple_weight)
    return loss[0, 0], pos, ns
